# baseline (speedup 1.0000x reference)
import functools

import jax
import jax.numpy as jnp
from jax import lax
from jax.experimental import pallas as pl
from jax.experimental.pallas import tpu as pltpu
from jax.experimental.pallas import tpu_sc as plsc

_NC = 2
_NS = 16
_NW = _NC * _NS
_C = 4
_NBUF = 3


def kernel(x, emb):
    (B,) = x.shape
    V, D = emb.shape
    bpw = B // _NW
    nchunk = bpw // _C

    x2 = x.reshape(_NW, nchunk, _C).astype(jnp.int32)

    mesh = plsc.VectorSubcoreMesh(core_axis_name="c", subcore_axis_name="s")

    @functools.partial(
        pl.kernel,
        out_type=jax.ShapeDtypeStruct((B, D), emb.dtype),
        mesh=mesh,
        scratch_types=[
            pltpu.VMEM((nchunk, _C), jnp.int32),
        ]
        + [pltpu.VMEM((_C, D), emb.dtype) for _ in range(_NBUF)]
        + [pltpu.SemaphoreType.DMA for _ in range(2 * _NBUF)],
    )
    def gather_k(x_hbm, emb_hbm, out_hbm, idx_v, *rest):
        bufs = rest[:_NBUF]
        gsem = rest[_NBUF : 2 * _NBUF]
        wsem = rest[2 * _NBUF :]
        wid = lax.axis_index("s") * _NC + lax.axis_index("c")
        rbase = wid * bpw
        pltpu.sync_copy(x_hbm.at[wid], idx_v)

        def fire_g(g, b):
            pltpu.async_copy(emb_hbm.at[idx_v.at[g]], bufs[b], gsem[b])

        def wait_g(g, b):
            pltpu.make_async_copy(emb_hbm.at[idx_v.at[g]], bufs[b], gsem[b]).wait()

        def fire_w(g, b):
            pltpu.async_copy(bufs[b], out_hbm.at[pl.ds(rbase + g * _C, _C)], wsem[b])

        def wait_w(g, b):
            pltpu.make_async_copy(
                bufs[b], out_hbm.at[pl.ds(rbase + g * _C, _C)], wsem[b]
            ).wait()

        def slot(g, b, first, last):
            if not first:
                pl.when(g >= 2)(lambda: wait_w(g - 2, (b + 1) % _NBUF))
            if not last:
                fire_g(g + 1, (b + 1) % _NBUF)
            wait_g(g, b)
            fire_w(g, b)

        fire_g(0, 0)

        nmain = (nchunk - 2) // _NBUF * _NBUF

        @pl.loop(0, nmain, step=_NBUF)
        def _(j):
            for k in range(_NBUF):
                g = j + k
                slot(g, k, first=False, last=False)

        for g in range(nmain, nchunk):
            b = g % _NBUF
            if g >= 2:
                wait_w(g - 2, (b + 1) % _NBUF)
            if g + 1 < nchunk:
                fire_g(g + 1, (g + 1) % _NBUF)
            wait_g(g, b)
            fire_w(g, b)

        wait_w(nchunk - 2, (nchunk - 2) % _NBUF)
        wait_w(nchunk - 1, (nchunk - 1) % _NBUF)

    return gather_k(x2, emb)
